# baseline (speedup 1.0000x reference)
import functools

import jax
import jax.numpy as jnp
from jax import lax
from jax.experimental import pallas as pl
from jax.experimental.pallas import tpu as pltpu
from jax.experimental.pallas import tpu_sc as plsc

_NC = 2
_NS = 16
_L = 16
_NW = _NC * _NS

_B = 16384
_D = 64
_BPW = _B // _NW
_CHUNK = 128
_GROUPS = _BPW // _L


def _fold_body(w1_ref, b1_ref, w2_ref, b2_ref, w3_ref, b3_ref,
               weff_ref, beff_ref):
    w2v = w2_ref[...]
    w3v = w3_ref[...]
    w23 = jnp.dot(w2v, w3v, preferred_element_type=jnp.float32)
    weff_ref[...] = jnp.dot(w1_ref[...], w23,
                            preferred_element_type=jnp.float32)
    beff_ref[...] = (jnp.dot(b1_ref[...], w23,
                             preferred_element_type=jnp.float32)
                     + jnp.dot(b2_ref[...], w3v,
                               preferred_element_type=jnp.float32)
                     + b3_ref[...])


_fold = pl.pallas_call(
    _fold_body,
    out_shape=(jax.ShapeDtypeStruct((2 * _D, 1), jnp.float32),
               jax.ShapeDtypeStruct((1, 1), jnp.float32)),
)


def _sc_body(tidx_hbm, cidx_hbm, ttab_hbm, ctab_hbm, w_hbm, b_hbm,
             out_hbm,
             tidx_v, cidx_v, trows_v, crows_v, w_v, b_v, out_v, sem):
    wid = lax.axis_index("s") * _NC + lax.axis_index("c")
    base = wid * _BPW

    pltpu.sync_copy(tidx_hbm.at[pl.ds(base, _BPW)], tidx_v)
    pltpu.sync_copy(cidx_hbm.at[pl.ds(base, _BPW)], cidx_v)
    copies = []
    for j in range(_BPW // _CHUNK):
        sl = pl.ds(j * _CHUNK, _CHUNK)
        copies.append(pltpu.async_copy(ttab_hbm.at[tidx_v.at[sl]],
                                       trows_v.at[sl], sem))
        copies.append(pltpu.async_copy(ctab_hbm.at[cidx_v.at[sl]],
                                       crows_v.at[sl], sem))
    pltpu.sync_copy(w_hbm, w_v)
    pltpu.sync_copy(b_hbm, b_v)
    for c in copies:
        c.wait()

    wvecs = [w_v[pl.ds(i * _L, _L)] for i in range(2 * _D // _L)]
    bvec = b_v[...]

    def group(g, carry):
        rows = g * _L + lax.iota(jnp.int32, _L)
        acc = jnp.zeros((_L,), jnp.float32)
        for j in range(_D):
            col = jnp.full((_L,), j, jnp.int32)
            acc = acc + plsc.load_gather(trows_v, [rows, col]) * wvecs[j // _L][j % _L]
        for j in range(_D):
            col = jnp.full((_L,), j, jnp.int32)
            acc = acc + (plsc.load_gather(crows_v, [rows, col])
                         * wvecs[(_D + j) // _L][j % _L])
        x = acc + bvec
        out_v[pl.ds(g * _L, _L)] = 1.0 / (1.0 + jnp.exp(-x))
        return carry

    lax.fori_loop(0, _GROUPS, group, 0)
    pltpu.sync_copy(out_v, out_hbm.at[pl.ds(base, _BPW)])


_sc_lookup = functools.partial(
    pl.kernel,
    mesh=plsc.VectorSubcoreMesh(core_axis_name="c", subcore_axis_name="s"),
    out_type=jax.ShapeDtypeStruct((_B,), jnp.float32),
    compiler_params=pltpu.CompilerParams(needs_layout_passes=False,
                                         use_tc_tiling_on_sc=False),
    scratch_types=[
        pltpu.VMEM((_BPW,), jnp.int32),
        pltpu.VMEM((_BPW,), jnp.int32),
        pltpu.VMEM((_BPW, _D), jnp.float32),
        pltpu.VMEM((_BPW, _D), jnp.float32),
        pltpu.VMEM((2 * _D,), jnp.float32),
        pltpu.VMEM((_L,), jnp.float32),
        pltpu.VMEM((_BPW,), jnp.float32),
        pltpu.SemaphoreType.DMA,
    ],
)(_sc_body)


def kernel(inputs, target_table, context_table, W1, b1, W2, b2, W3, b3):
    tgt = inputs[:, 0]
    ctx = inputs[:, 1]
    weff, beff = _fold(W1, b1.reshape(1, -1), W2, b2.reshape(1, -1),
                       W3, b3.reshape(1, 1))
    w1d = weff.reshape(-1)
    bpad = jnp.tile(beff.reshape(-1), _L)
    out = _sc_lookup(tgt, ctx, target_table, context_table, w1d, bpad)
    return out.reshape(_B, 1)


# speedup vs baseline: 5.6905x; 5.6905x over previous
import functools

import jax
import jax.numpy as jnp
from jax import lax
from jax.experimental import pallas as pl
from jax.experimental.pallas import tpu as pltpu
from jax.experimental.pallas import tpu_sc as plsc

_NC = 2
_NS = 16
_L = 16
_NW = _NC * _NS

_B = 16384
_D = 64
_V = 1000000
_BPW = _B // _NW
_CHUNK = 128
_GROUPS = _BPW // _L
_BLK = 8192


def _fold_body(w1_ref, b1_ref, w2_ref, b2_ref, w3_ref, b3_ref,
               weff_ref, beff_ref):
    w2v = w2_ref[...]
    w3v = w3_ref[...]
    w23 = jnp.dot(w2v, w3v, preferred_element_type=jnp.float32)
    weff_ref[...] = jnp.dot(w1_ref[...], w23,
                            preferred_element_type=jnp.float32)
    beff_ref[...] = (jnp.dot(b1_ref[...], w23,
                             preferred_element_type=jnp.float32)
                     + jnp.dot(b2_ref[...], w3v,
                               preferred_element_type=jnp.float32)
                     + b3_ref[...])


_fold = pl.pallas_call(
    _fold_body,
    out_shape=(jax.ShapeDtypeStruct((2 * _D, 1), jnp.float32),
               jax.ShapeDtypeStruct((1, 1), jnp.float32)),
)


def _proj_body(w_ref, t_ref, c_ref, pt_ref, pc_ref):
    w = w_ref[...]
    wt = w[:, :_D]
    wc = w[:, _D:]
    pt_ref[...] = jnp.dot(wt, t_ref[...],
                          preferred_element_type=jnp.float32)[0]
    pc_ref[...] = jnp.dot(wc, c_ref[...],
                          preferred_element_type=jnp.float32)[0]


_proj = pl.pallas_call(
    _proj_body,
    grid=(pl.cdiv(_V, _BLK),),
    in_specs=[
        pl.BlockSpec((1, 2 * _D), lambda i: (0, 0)),
        pl.BlockSpec((_D, _BLK), lambda i: (0, i)),
        pl.BlockSpec((_D, _BLK), lambda i: (0, i)),
    ],
    out_specs=[
        pl.BlockSpec((_BLK,), lambda i: (i,)),
        pl.BlockSpec((_BLK,), lambda i: (i,)),
    ],
    out_shape=(jax.ShapeDtypeStruct((_V,), jnp.float32),
               jax.ShapeDtypeStruct((_V,), jnp.float32)),
)


def _sc_body(tidx_hbm, cidx_hbm, pt_hbm, pc_hbm, b_hbm,
             out_hbm,
             tidx_v, cidx_v, gt_v, gc_v, b_v, out_v, sem):
    wid = lax.axis_index("s") * _NC + lax.axis_index("c")
    base = wid * _BPW

    pltpu.sync_copy(tidx_hbm.at[pl.ds(base, _BPW)], tidx_v)
    pltpu.sync_copy(cidx_hbm.at[pl.ds(base, _BPW)], cidx_v)
    copies = []
    for j in range(_BPW // _CHUNK):
        sl = pl.ds(j * _CHUNK, _CHUNK)
        copies.append(pltpu.async_copy(pt_hbm.at[tidx_v.at[sl]],
                                       gt_v.at[sl], sem))
        copies.append(pltpu.async_copy(pc_hbm.at[cidx_v.at[sl]],
                                       gc_v.at[sl], sem))
    pltpu.sync_copy(b_hbm, b_v)
    for c in copies:
        c.wait()

    bvec = b_v[...]

    def group(g, carry):
        sl = pl.ds(g * _L, _L)
        x = gt_v[sl] + gc_v[sl] + bvec
        out_v[sl] = 1.0 / (1.0 + jnp.exp(-x))
        return carry

    lax.fori_loop(0, _GROUPS, group, 0)
    pltpu.sync_copy(out_v, out_hbm.at[pl.ds(base, _BPW)])


_sc_lookup = functools.partial(
    pl.kernel,
    mesh=plsc.VectorSubcoreMesh(core_axis_name="c", subcore_axis_name="s"),
    out_type=jax.ShapeDtypeStruct((_B,), jnp.float32),
    compiler_params=pltpu.CompilerParams(needs_layout_passes=False,
                                         use_tc_tiling_on_sc=False),
    scratch_types=[
        pltpu.VMEM((_BPW,), jnp.int32),
        pltpu.VMEM((_BPW,), jnp.int32),
        pltpu.VMEM((_BPW,), jnp.float32),
        pltpu.VMEM((_BPW,), jnp.float32),
        pltpu.VMEM((_L,), jnp.float32),
        pltpu.VMEM((_BPW,), jnp.float32),
        pltpu.SemaphoreType.DMA,
    ],
)(_sc_body)


def kernel(inputs, target_table, context_table, W1, b1, W2, b2, W3, b3):
    tgt = inputs[:, 0]
    ctx = inputs[:, 1]
    weff, beff = _fold(W1, b1.reshape(1, -1), W2, b2.reshape(1, -1),
                       W3, b3.reshape(1, 1))
    proj_t, proj_c = _proj(weff.reshape(1, -1),
                           target_table.T, context_table.T)
    bsplat = jnp.tile(beff.reshape(-1), _L)
    out = _sc_lookup(tgt, ctx, proj_t, proj_c, bsplat)
    return out.reshape(_B, 1)


# speedup vs baseline: 6.3222x; 1.1110x over previous
import functools

import jax
import jax.numpy as jnp
from jax import lax
from jax.experimental import pallas as pl
from jax.experimental.pallas import tpu as pltpu
from jax.experimental.pallas import tpu_sc as plsc

_NC = 2
_NS = 16
_L = 16
_NW = _NC * _NS

_B = 16384
_D = 64
_V = 1000000
_BPW = _B // _NW
_CHUNK = 128
_GROUPS = _BPW // _L
_BLK = 16384


def _fold_body(w1_ref, b1_ref, w2_ref, b2_ref, w3_ref, b3_ref,
               weff_ref, beff_ref):
    w2v = w2_ref[...]
    w3v = w3_ref[...]
    w23 = jnp.dot(w2v, w3v, preferred_element_type=jnp.float32)
    weff_ref[...] = jnp.dot(w1_ref[...], w23,
                            preferred_element_type=jnp.float32)
    beff_ref[...] = (jnp.dot(b1_ref[...], w23,
                             preferred_element_type=jnp.float32)
                     + jnp.dot(b2_ref[...], w3v,
                               preferred_element_type=jnp.float32)
                     + b3_ref[...])


_fold = pl.pallas_call(
    _fold_body,
    out_shape=(jax.ShapeDtypeStruct((2 * _D, 1), jnp.float32),
               jax.ShapeDtypeStruct((1, 1), jnp.float32)),
)


def _proj_body(w_ref, t_ref, c_ref, pt_ref, pc_ref):
    w = w_ref[...]
    wt = w[:, :_D]
    wc = w[:, _D:]
    pt_ref[...] = jnp.dot(wt, t_ref[...],
                          preferred_element_type=jnp.float32)[0]
    pc_ref[...] = jnp.dot(wc, c_ref[...],
                          preferred_element_type=jnp.float32)[0]


_proj = pl.pallas_call(
    _proj_body,
    grid=(pl.cdiv(_V, _BLK),),
    in_specs=[
        pl.BlockSpec((1, 2 * _D), lambda i: (0, 0)),
        pl.BlockSpec((_D, _BLK), lambda i: (0, i)),
        pl.BlockSpec((_D, _BLK), lambda i: (0, i)),
    ],
    out_specs=[
        pl.BlockSpec((_BLK,), lambda i: (i,)),
        pl.BlockSpec((_BLK,), lambda i: (i,)),
    ],
    out_shape=(jax.ShapeDtypeStruct((_V,), jnp.float32),
               jax.ShapeDtypeStruct((_V,), jnp.float32)),
)


def _sc_body(tidx_hbm, cidx_hbm, pt_hbm, pc_hbm, b_hbm,
             out_hbm,
             tidx_v, cidx_v, gt_v, gc_v, b_v, out_v, sem):
    wid = lax.axis_index("s") * _NC + lax.axis_index("c")
    base = wid * _BPW

    pltpu.sync_copy(tidx_hbm.at[pl.ds(base, _BPW)], tidx_v)
    pltpu.sync_copy(cidx_hbm.at[pl.ds(base, _BPW)], cidx_v)
    copies = []
    for j in range(_BPW // _CHUNK):
        sl = pl.ds(j * _CHUNK, _CHUNK)
        copies.append(pltpu.async_copy(pt_hbm.at[tidx_v.at[sl]],
                                       gt_v.at[sl], sem))
        copies.append(pltpu.async_copy(pc_hbm.at[cidx_v.at[sl]],
                                       gc_v.at[sl], sem))
    pltpu.sync_copy(b_hbm, b_v)
    for c in copies:
        c.wait()

    bvec = b_v[...]

    def group(g, carry):
        sl = pl.ds(g * _L, _L)
        x = gt_v[sl] + gc_v[sl] + bvec
        out_v[sl] = 1.0 / (1.0 + jnp.exp(-x))
        return carry

    lax.fori_loop(0, _GROUPS, group, 0)
    pltpu.sync_copy(out_v, out_hbm.at[pl.ds(base, _BPW)])


_sc_lookup = functools.partial(
    pl.kernel,
    mesh=plsc.VectorSubcoreMesh(core_axis_name="c", subcore_axis_name="s"),
    out_type=jax.ShapeDtypeStruct((_B,), jnp.float32),
    compiler_params=pltpu.CompilerParams(needs_layout_passes=False,
                                         use_tc_tiling_on_sc=False),
    scratch_types=[
        pltpu.VMEM((_BPW,), jnp.int32),
        pltpu.VMEM((_BPW,), jnp.int32),
        pltpu.VMEM((_BPW,), jnp.float32),
        pltpu.VMEM((_BPW,), jnp.float32),
        pltpu.VMEM((_L,), jnp.float32),
        pltpu.VMEM((_BPW,), jnp.float32),
        pltpu.SemaphoreType.DMA,
    ],
)(_sc_body)


def kernel(inputs, target_table, context_table, W1, b1, W2, b2, W3, b3):
    tgt = inputs[:, 0]
    ctx = inputs[:, 1]
    weff, beff = _fold(W1, b1.reshape(1, -1), W2, b2.reshape(1, -1),
                       W3, b3.reshape(1, 1))
    proj_t, proj_c = _proj(weff.reshape(1, -1),
                           target_table.T, context_table.T)
    bsplat = jnp.tile(beff.reshape(-1), _L)
    out = _sc_lookup(tgt, ctx, proj_t, proj_c, bsplat)
    return out.reshape(_B, 1)
